# Initial kernel scaffold; baseline (speedup 1.0000x reference)
import jax
import jax.numpy as jnp
from jax.experimental import pallas as pl


def kernel(instruction, table, ln_gamma, ln_beta):
    raise NotImplementedError("write your pallas kernel here")


# speedup vs baseline: 1.3886x; 1.3886x over previous
import jax
import jax.numpy as jnp
import numpy as np
from jax import lax
from jax.experimental import pallas as pl
from jax.experimental.pallas import tpu as pltpu
from jax.experimental.pallas import tpu_sc as plsc

N_INP = 100000
EMBED_DIM = 64
LN_EPS = 1e-5
B, L = 1024, 200
N_ROWS = B * L

NUM_WORKERS = 32
ROWS_PER_WORKER = N_ROWS // NUM_WORKERS
CHUNK = 128
CHUNKS_PER_WORKER = ROWS_PER_WORKER // CHUNK
ROW_UNROLL = 4


def _pos_encoding_table(seq_len, channels):
    ch = int(np.ceil(channels / 2) * 2)
    inv_freq = 1.0 / (10000.0 ** (np.arange(0, ch, 2, dtype=np.float32) / ch))
    pos = np.arange(seq_len, dtype=np.float32)
    sin_inp = pos[:, None] * inv_freq[None, :]
    emb = np.stack((np.sin(sin_inp), np.cos(sin_inp)), axis=-1).reshape(seq_len, ch)
    return jnp.asarray(emb[:, :channels], dtype=jnp.float32)


def _rsqrt_vec(t):
    i = plsc.bitcast(t, jnp.int32)
    y = plsc.bitcast(jnp.int32(0x5F3759DF) - (i >> 1), jnp.float32)
    half_t = 0.5 * t
    y = y * (1.5 - half_t * y * y)
    y = y * (1.5 - half_t * y * y)
    y = y * (1.5 - half_t * y * y)
    return y


def _sc_body(instr_hbm, table_hbm, gamma_hbm, beta_hbm, pe_hbm, out_hbm,
             idx_v, rows_v, pe_v, gamma_v, beta_v, sem):
    wid = lax.axis_index("s") * 2 + lax.axis_index("c")
    base = wid * ROWS_PER_WORKER

    pltpu.sync_copy(pe_hbm, pe_v)
    pltpu.sync_copy(gamma_hbm, gamma_v)
    pltpu.sync_copy(beta_hbm, beta_v)

    gvecs = [gamma_v[pl.ds(k * 16, 16)] for k in range(4)]
    bvecs = [beta_v[pl.ds(k * 16, 16)] for k in range(4)]
    inv_d = jnp.float32(1.0 / EMBED_DIM)

    def one_row(start, r):
        lr = (start + r) % L
        v = [rows_v[r, pl.ds(k * 16, 16)] + pe_v[lr, pl.ds(k * 16, 16)]
             for k in range(4)]
        svec = (v[0] + v[1]) + (v[2] + v[3])
        qvec = v[0] * v[0] + v[1] * v[1] + v[2] * v[2] + v[3] * v[3]
        mean = jnp.sum(svec) * inv_d
        ex2 = jnp.sum(qvec) * inv_d
        t = (ex2 - mean * mean) + LN_EPS
        tv = jnp.full((16,), t, jnp.float32)
        a = _rsqrt_vec(tv)
        b = jnp.full((16,), -mean, jnp.float32) * a
        for k in range(4):
            y = (v[k] * a + b) * gvecs[k] + bvecs[k]
            rows_v[r, pl.ds(k * 16, 16)] = y

    def chunk_body(c, _):
        start = base + c * CHUNK
        pltpu.sync_copy(instr_hbm.at[pl.ds(start, CHUNK)], idx_v)
        pltpu.async_copy(table_hbm.at[idx_v], rows_v, sem).wait()

        def row_body(rb, _):
            for u in range(ROW_UNROLL):
                one_row(start, rb * ROW_UNROLL + u)
            return _

        lax.fori_loop(0, CHUNK // ROW_UNROLL, row_body, None)

        pltpu.sync_copy(rows_v, out_hbm.at[pl.ds(start, CHUNK)])
        return _

    lax.fori_loop(0, CHUNKS_PER_WORKER, chunk_body, None)


def kernel(instruction, table, ln_gamma, ln_beta):
    instr_flat = instruction.reshape(N_ROWS).astype(jnp.int32)
    pe = _pos_encoding_table(L, EMBED_DIM)

    func = pl.kernel(
        _sc_body,
        out_type=jax.ShapeDtypeStruct((N_ROWS, EMBED_DIM), jnp.float32),
        mesh=plsc.VectorSubcoreMesh(core_axis_name="c", subcore_axis_name="s"),
        scratch_types=[
            pltpu.VMEM((CHUNK,), jnp.int32),
            pltpu.VMEM((CHUNK, EMBED_DIM), jnp.float32),
            pltpu.VMEM((L, EMBED_DIM), jnp.float32),
            pltpu.VMEM((EMBED_DIM,), jnp.float32),
            pltpu.VMEM((EMBED_DIM,), jnp.float32),
            pltpu.SemaphoreType.DMA,
        ],
        compiler_params=pltpu.CompilerParams(
            needs_layout_passes=False, use_tc_tiling_on_sc=False),
    )
    out = func(instr_flat, table, ln_gamma, ln_beta, pe)
    return out.reshape(B, L, EMBED_DIM)


# speedup vs baseline: 1.6234x; 1.1691x over previous
import jax
import jax.numpy as jnp
import numpy as np
from jax import lax
from jax.experimental import pallas as pl
from jax.experimental.pallas import tpu as pltpu
from jax.experimental.pallas import tpu_sc as plsc

N_INP = 100000
EMBED_DIM = 64
LN_EPS = 1e-5
B, L = 1024, 200
N_ROWS = B * L

NUM_WORKERS = 32
ROWS_PER_WORKER = N_ROWS // NUM_WORKERS
CHUNK = 128
NCHUNKS = ROWS_PER_WORKER // CHUNK
ROW_UNROLL = 8


def _pos_encoding_table(seq_len, channels):
    ch = int(np.ceil(channels / 2) * 2)
    inv_freq = 1.0 / (10000.0 ** (np.arange(0, ch, 2, dtype=np.float32) / ch))
    pos = np.arange(seq_len, dtype=np.float32)
    sin_inp = pos[:, None] * inv_freq[None, :]
    emb = np.stack((np.sin(sin_inp), np.cos(sin_inp)), axis=-1).reshape(seq_len, ch)
    return jnp.asarray(emb[:, :channels], dtype=jnp.float32)


def _rsqrt_vec(t):
    i = plsc.bitcast(t, jnp.int32)
    y = plsc.bitcast(jnp.int32(0x5F3759DF) - (i >> 1), jnp.float32)
    half_t = 0.5 * t
    y = y * (1.5 - half_t * y * y)
    y = y * (1.5 - half_t * y * y)
    y = y * (1.5 - half_t * y * y)
    return y


def _sc_body(instr_hbm, table_hbm, gamma_hbm, beta_hbm, pe_hbm, out_hbm,
             idx_all, rows0, rows1, out0, out1, pe_v, gamma_v, beta_v,
             sg0, sg1, sw0, sw1):
    wid = lax.axis_index("s") * 2 + lax.axis_index("c")
    base = wid * ROWS_PER_WORKER

    pltpu.sync_copy(pe_hbm, pe_v)
    pltpu.sync_copy(gamma_hbm, gamma_v)
    pltpu.sync_copy(beta_hbm, beta_v)
    pltpu.sync_copy(instr_hbm.at[pl.ds(base, ROWS_PER_WORKER)], idx_all)

    rows = [rows0, rows1]
    outs = [out0, out1]
    sg = [sg0, sg1]
    sw = [sw0, sw1]

    gvecs = [gamma_v[pl.ds(k * 16, 16)] for k in range(4)]
    bvecs = [beta_v[pl.ds(k * 16, 16)] for k in range(4)]
    inv_d = jnp.float32(1.0 / EMBED_DIM)

    def gather_issue(c, b):
        off = pl.multiple_of(c * CHUNK, CHUNK)
        pltpu.async_copy(
            table_hbm.at[idx_all.at[pl.ds(off, CHUNK)]], rows[b], sg[b])

    def gather_wait(b):
        pltpu.make_async_copy(
            table_hbm.at[idx_all.at[pl.ds(0, CHUNK)]], rows[b], sg[b]).wait()

    def wout_issue(c, b):
        off = pl.multiple_of(base + c * CHUNK, CHUNK)
        pltpu.async_copy(outs[b], out_hbm.at[pl.ds(off, CHUNK)], sw[b])

    def wout_wait(b):
        pltpu.make_async_copy(
            outs[b], out_hbm.at[pl.ds(0, CHUNK)], sw[b]).wait()

    def one_row(start, r, rv, ov):
        lr = (start + r) % L
        v = [rv[r, pl.ds(k * 16, 16)] + pe_v[lr, pl.ds(k * 16, 16)]
             for k in range(4)]
        svec = (v[0] + v[1]) + (v[2] + v[3])
        qvec = v[0] * v[0] + v[1] * v[1] + v[2] * v[2] + v[3] * v[3]
        mean = jnp.sum(svec) * inv_d
        ex2 = jnp.sum(qvec) * inv_d
        t = (ex2 - mean * mean) + LN_EPS
        tv = jnp.full((16,), t, jnp.float32)
        a = _rsqrt_vec(tv)
        bvec = jnp.full((16,), -mean, jnp.float32) * a
        for k in range(4):
            ov[r, pl.ds(k * 16, 16)] = (v[k] * a + bvec) * gvecs[k] + bvecs[k]

    def compute(c, b):
        start = base + c * CHUNK
        rv, ov = rows[b], outs[b]

        def row_body(rb, _):
            for u in range(ROW_UNROLL):
                one_row(start, rb * ROW_UNROLL + u, rv, ov)
            return _

        lax.fori_loop(0, CHUNK // ROW_UNROLL, row_body, None)

    gather_issue(0, 0)
    gather_issue(1, 1)

    for b in range(2):
        gather_wait(b)
        compute(b, b)
        gather_issue(b + 2, b)
        wout_issue(b, b)

    def it_body(it, _):
        for b in range(2):
            c = it * 2 + b
            gather_wait(b)
            wout_wait(b)
            compute(c, b)
            gather_issue(c + 2, b)
            wout_issue(c, b)
        return _

    lax.fori_loop(1, NCHUNKS // 2 - 1, it_body, None)

    for b in range(2):
        c = NCHUNKS - 2 + b
        gather_wait(b)
        wout_wait(b)
        compute(c, b)
        wout_issue(c, b)

    for b in range(2):
        wout_wait(b)


def kernel(instruction, table, ln_gamma, ln_beta):
    instr_flat = instruction.reshape(N_ROWS).astype(jnp.int32)
    pe = _pos_encoding_table(L, EMBED_DIM)

    func = pl.kernel(
        _sc_body,
        out_type=jax.ShapeDtypeStruct((N_ROWS, EMBED_DIM), jnp.float32),
        mesh=plsc.VectorSubcoreMesh(core_axis_name="c", subcore_axis_name="s"),
        scratch_types=[
            pltpu.VMEM((ROWS_PER_WORKER,), jnp.int32),
            pltpu.VMEM((CHUNK, EMBED_DIM), jnp.float32),
            pltpu.VMEM((CHUNK, EMBED_DIM), jnp.float32),
            pltpu.VMEM((CHUNK, EMBED_DIM), jnp.float32),
            pltpu.VMEM((CHUNK, EMBED_DIM), jnp.float32),
            pltpu.VMEM((L, EMBED_DIM), jnp.float32),
            pltpu.VMEM((EMBED_DIM,), jnp.float32),
            pltpu.VMEM((EMBED_DIM,), jnp.float32),
            pltpu.SemaphoreType.DMA,
            pltpu.SemaphoreType.DMA,
            pltpu.SemaphoreType.DMA,
            pltpu.SemaphoreType.DMA,
        ],
        compiler_params=pltpu.CompilerParams(
            needs_layout_passes=False, use_tc_tiling_on_sc=False),
    )
    out = func(instr_flat, table, ln_gamma, ln_beta, pe)
    return out.reshape(B, L, EMBED_DIM)


# speedup vs baseline: 2.8581x; 1.7605x over previous
import jax
import jax.numpy as jnp
import numpy as np
from jax import lax
from jax.experimental import pallas as pl
from jax.experimental.pallas import tpu as pltpu
from jax.experimental.pallas import tpu_sc as plsc

N_INP = 100000
EMBED_DIM = 64
LN_EPS = 1e-5
B, L = 1024, 200
N_ROWS = B * L

NUM_WORKERS = 32
ROWS_PER_WORKER = N_ROWS // NUM_WORKERS
CHUNK = 128
NCHUNKS = ROWS_PER_WORKER // CHUNK
ROW_UNROLL = 8


def _pos_encoding_table(seq_len, channels):
    ch = int(np.ceil(channels / 2) * 2)
    inv_freq = 1.0 / (10000.0 ** (np.arange(0, ch, 2, dtype=np.float32) / ch))
    pos = np.arange(seq_len, dtype=np.float32)
    sin_inp = pos[:, None] * inv_freq[None, :]
    emb = np.stack((np.sin(sin_inp), np.cos(sin_inp)), axis=-1).reshape(seq_len, ch)
    return jnp.asarray(emb[:, :channels], dtype=jnp.float32)


def _rsqrt_vec(t):
    i = plsc.bitcast(t, jnp.int32)
    y = plsc.bitcast(jnp.int32(0x5F3759DF) - (i >> 1), jnp.float32)
    half_t = 0.5 * t
    y = y * (1.5 - half_t * y * y)
    y = y * (1.5 - half_t * y * y)
    return y


def _sc_body(instr_hbm, table_hbm, gamma_hbm, beta_hbm, pe_hbm, out_hbm,
             idx_all, rows0, rows1, out0, out1, pe_v, gamma_v, beta_v,
             sg0, sg1, sw0, sw1):
    wid = lax.axis_index("s") * 2 + lax.axis_index("c")
    base = wid * ROWS_PER_WORKER

    pltpu.sync_copy(pe_hbm, pe_v)
    pltpu.sync_copy(gamma_hbm, gamma_v)
    pltpu.sync_copy(beta_hbm, beta_v)
    pltpu.sync_copy(instr_hbm.at[pl.ds(base, ROWS_PER_WORKER)], idx_all)

    rows = [rows0, rows1]
    outs = [out0, out1]
    sg = [sg0, sg1]
    sw = [sw0, sw1]

    gvecs = [gamma_v[pl.ds(k * 16, 16)] for k in range(4)]
    bvecs = [beta_v[pl.ds(k * 16, 16)] for k in range(4)]
    inv_d = jnp.float32(1.0 / EMBED_DIM)

    def gather_issue(c, b):
        off = pl.multiple_of(c * CHUNK, CHUNK)
        pltpu.async_copy(
            table_hbm.at[idx_all.at[pl.ds(off, CHUNK)]], rows[b], sg[b])

    def gather_wait(b):
        pltpu.make_async_copy(
            table_hbm.at[idx_all.at[pl.ds(0, CHUNK)]], rows[b], sg[b]).wait()

    def wout_issue(c, b):
        off = pl.multiple_of(base + c * CHUNK, CHUNK)
        pltpu.async_copy(outs[b], out_hbm.at[pl.ds(off, CHUNK)], sw[b])

    def wout_wait(b):
        pltpu.make_async_copy(
            outs[b], out_hbm.at[pl.ds(0, CHUNK)], sw[b]).wait()

    perms = [lax.iota(jnp.int32, 16) ^ k for k in (1, 2, 4, 8)]

    def one_row(start, r, rv, ov):
        lr = (start + r) % L
        v = [rv[r, pl.ds(k * 16, 16)] + pe_v[lr, pl.ds(k * 16, 16)]
             for k in range(4)]
        s = (v[0] + v[1]) + (v[2] + v[3])
        q = (v[0] * v[0] + v[1] * v[1]) + (v[2] * v[2] + v[3] * v[3])
        for p in perms:
            s = s + s.at[p].get(mode="promise_in_bounds")
            q = q + q.at[p].get(mode="promise_in_bounds")
        mean = s * inv_d
        t = q * inv_d - mean * mean + LN_EPS
        a = _rsqrt_vec(t)
        cmean = mean * a
        for k in range(4):
            ov[r, pl.ds(k * 16, 16)] = (v[k] * a - cmean) * gvecs[k] + bvecs[k]

    def compute(c, b):
        start = base + c * CHUNK
        rv, ov = rows[b], outs[b]

        @plsc.parallel_loop(0, CHUNK, 1, unroll=ROW_UNROLL)
        def _(r):
            one_row(start, r, rv, ov)

    gather_issue(0, 0)
    gather_issue(1, 1)

    for b in range(2):
        gather_wait(b)
        compute(b, b)
        gather_issue(b + 2, b)
        wout_issue(b, b)

    def it_body(it, _):
        for b in range(2):
            c = it * 2 + b
            gather_wait(b)
            wout_wait(b)
            compute(c, b)
            gather_issue(c + 2, b)
            wout_issue(c, b)
        return _

    lax.fori_loop(1, NCHUNKS // 2 - 1, it_body, None)

    for b in range(2):
        c = NCHUNKS - 2 + b
        gather_wait(b)
        wout_wait(b)
        compute(c, b)
        wout_issue(c, b)

    for b in range(2):
        wout_wait(b)


def kernel(instruction, table, ln_gamma, ln_beta):
    instr_flat = instruction.reshape(N_ROWS).astype(jnp.int32)
    pe = _pos_encoding_table(L, EMBED_DIM)

    func = pl.kernel(
        _sc_body,
        out_type=jax.ShapeDtypeStruct((N_ROWS, EMBED_DIM), jnp.float32),
        mesh=plsc.VectorSubcoreMesh(core_axis_name="c", subcore_axis_name="s"),
        scratch_types=[
            pltpu.VMEM((ROWS_PER_WORKER,), jnp.int32),
            pltpu.VMEM((CHUNK, EMBED_DIM), jnp.float32),
            pltpu.VMEM((CHUNK, EMBED_DIM), jnp.float32),
            pltpu.VMEM((CHUNK, EMBED_DIM), jnp.float32),
            pltpu.VMEM((CHUNK, EMBED_DIM), jnp.float32),
            pltpu.VMEM((L, EMBED_DIM), jnp.float32),
            pltpu.VMEM((EMBED_DIM,), jnp.float32),
            pltpu.VMEM((EMBED_DIM,), jnp.float32),
            pltpu.SemaphoreType.DMA,
            pltpu.SemaphoreType.DMA,
            pltpu.SemaphoreType.DMA,
            pltpu.SemaphoreType.DMA,
        ],
        compiler_params=pltpu.CompilerParams(
            needs_layout_passes=False, use_tc_tiling_on_sc=False),
    )
    out = func(instr_flat, table, ln_gamma, ln_beta, pe)
    return out.reshape(B, L, EMBED_DIM)
